# baseline (speedup 1.0000x reference)
import functools

import jax
import jax.numpy as jnp
from jax import lax
from jax.experimental import pallas as pl
from jax.experimental.pallas import tpu as pltpu
from jax.experimental.pallas import tpu_sc as plsc

_VOCAB = 100000
_D = 64
_B = 16384
_NEG = 3
_L = 16

_NC, _NS = 2, 16
_NW = _NC * _NS
_NROUND = _D // _NW
_CH = 2048
_NCH = _B // _CH
_ROWLEN = _B * 4


@functools.cache
def _build_sc_dim_dots():
    mesh = plsc.VectorSubcoreMesh(core_axis_name="c", subcore_axis_name="s")

    @functools.partial(
        pl.kernel,
        mesh=mesh,
        compiler_params=pltpu.CompilerParams(
            use_tc_tiling_on_sc=True, needs_layout_passes=False),
        out_type=jax.ShapeDtypeStruct((_D * _ROWLEN,), jnp.float32),
        scratch_types=[
            pltpu.VMEM((_VOCAB,), jnp.float32),
            pltpu.VMEM((_CH,), jnp.int32),
            pltpu.VMEM((_CH,), jnp.int32),
            pltpu.VMEM((_CH,), jnp.int32),
            pltpu.VMEM((_CH,), jnp.int32),
            pltpu.VMEM((_L,), jnp.int32),
            pltpu.VMEM((4 * _CH,), jnp.float32),
            pltpu.VMEM((4 * _CH,), jnp.float32),
            pltpu.SemaphoreType.DMA,
            pltpu.SemaphoreType.DMA,
            pltpu.SemaphoreType.DMA,
            pltpu.SemaphoreType.DMA,
            pltpu.SemaphoreType.DMA,
            pltpu.SemaphoreType.DMA,
            pltpu.SemaphoreType.DMA,
        ],
    )
    def _sc_dim_dots(embT_hbm, tidx_hbm, cidx_hbm, nidx_hbm, parts_out,
                     ed, tq0, tq1, cq0, cq1, niv, av0, av1,
                     sem_e, st0, st1, sc0, sc1, so0, so1):
        wid = lax.axis_index("s") * _NC + lax.axis_index("c")
        pltpu.sync_copy(nidx_hbm, niv)
        nvals = niv[...]
        tqs, cqs, avs = (tq0, tq1), (cq0, cq1), (av0, av1)
        sts, scs, sos = (st0, st1), (sc0, sc1), (so0, so1)

        def start_idx(k, buf):
            cp_t = pltpu.async_copy(
                tidx_hbm.at[pl.ds(k * _CH, _CH)], tqs[buf], sts[buf])
            cp_c = pltpu.async_copy(
                cidx_hbm.at[pl.ds(k * _CH, _CH)], cqs[buf], scs[buf])
            return cp_t, cp_c

        for r in range(_NROUND):
            d = r * _NW + wid
            pltpu.sync_copy(embT_hbm.at[d], ed)
            nd = plsc.load_gather(ed, [nvals])
            nb0 = jnp.broadcast_to(nd[0], (_L,))
            nb1 = jnp.broadcast_to(nd[1], (_L,))
            nb2 = jnp.broadcast_to(nd[2], (_L,))
            row_base = d * _ROWLEN

            pend_idx = [start_idx(0, 0), None]
            pend_out = [None, None]
            for k in range(_NCH):
                buf = k & 1
                if k + 1 < _NCH:
                    pend_idx[1 - buf] = start_idx(k + 1, 1 - buf)
                cp_t, cp_c = pend_idx[buf]
                cp_t.wait()
                cp_c.wait()
                if pend_out[buf] is not None:
                    pend_out[buf].wait()
                tqi, cqi, accv = tqs[buf], cqs[buf], avs[buf]

                def group(jj, _, tqi=tqi, cqi=cqi, accv=accv,
                          nb0=nb0, nb1=nb1, nb2=nb2):
                    js = [jj * 8 + u for u in range(8)]
                    tvs = [tqi[pl.ds(j * _L, _L)] for j in js]
                    cvs = [cqi[pl.ds(j * _L, _L)] for j in js]
                    tcols = [plsc.load_gather(ed, [tv]) for tv in tvs]
                    ccols = [plsc.load_gather(ed, [cv]) for cv in cvs]
                    for u, j in enumerate(js):
                        accv[pl.ds(j * _L, _L)] = tcols[u] * ccols[u]
                    for u, j in enumerate(js):
                        accv[pl.ds(_CH + j * _L, _L)] = tcols[u] * nb0
                        accv[pl.ds(2 * _CH + j * _L, _L)] = tcols[u] * nb1
                        accv[pl.ds(3 * _CH + j * _L, _L)] = tcols[u] * nb2
                    return _

                lax.fori_loop(0, _CH // (_L * 8), group, None)
                pend_out[buf] = pltpu.async_copy(
                    accv,
                    parts_out.at[pl.ds(row_base + k * 4 * _CH, 4 * _CH)],
                    sos[buf])
            for po in pend_out:
                if po is not None:
                    po.wait()

    return _sc_dim_dots


_TCROWS = _ROWLEN // 128


def _tc_body(parts_ref, out_ref, acc_ref):
    i = pl.program_id(0)
    x = parts_ref[...]
    part = x[0:_TCROWS]
    for k in range(1, 8):
        part = part + x[k * _TCROWS:(k + 1) * _TCROWS]

    @pl.when(i == 0)
    def _():
        acc_ref[...] = jnp.zeros((_TCROWS, 128), jnp.float32)

    acc_ref[...] += part

    @pl.when(i == _D // 8 - 1)
    def _():
        s = acc_ref[...]
        ch = (lax.broadcasted_iota(jnp.int32, (_TCROWS, 128), 0)
              // (_CH // 128)) % 4
        s = jnp.where(ch == 0, s, -s)
        ls = jnp.minimum(s, 0.0) - jnp.log1p(jnp.exp(-jnp.abs(s)))
        out_ref[0, 0] = -jnp.sum(ls) / _B


def _tc_loss(parts):
    return pl.pallas_call(
        _tc_body,
        grid=(_D // 8,),
        in_specs=[
            pl.BlockSpec((8 * _TCROWS, 128), lambda i: (i, 0)),
        ],
        out_specs=pl.BlockSpec(memory_space=pltpu.SMEM),
        out_shape=jax.ShapeDtypeStruct((1, 1), jnp.float32),
        scratch_shapes=[pltpu.VMEM((_TCROWS, 128), jnp.float32)],
    )(parts.reshape(_D * _TCROWS, 128))


def kernel(target_idx, context_idx, embeddings, neg_idx):
    nidx = jnp.concatenate(
        [neg_idx.astype(jnp.int32), jnp.zeros((_L - _NEG,), jnp.int32)])
    parts = _build_sc_dim_dots()(
        embeddings.T, target_idx.astype(jnp.int32),
        context_idx.astype(jnp.int32), nidx)
    acc = _tc_loss(parts)
    return acc[0, 0]


# speedup vs baseline: 1.0243x; 1.0243x over previous
import functools

import jax
import jax.numpy as jnp
from jax import lax
from jax.experimental import pallas as pl
from jax.experimental.pallas import tpu as pltpu
from jax.experimental.pallas import tpu_sc as plsc

_VOCAB = 100000
_D = 64
_B = 16384
_NEG = 3
_L = 16

_NC, _NS = 2, 16
_NW = _NC * _NS
_NROUND = _D // _NW
_CH = 2048
_NCH = _B // _CH
_ROWLEN = _B * 4


@functools.cache
def _build_sc_dim_dots():
    mesh = plsc.VectorSubcoreMesh(core_axis_name="c", subcore_axis_name="s")

    @functools.partial(
        pl.kernel,
        mesh=mesh,
        compiler_params=pltpu.CompilerParams(
            use_tc_tiling_on_sc=True, needs_layout_passes=False),
        out_type=jax.ShapeDtypeStruct((_D * _ROWLEN,), jnp.float32),
        scratch_types=[
            pltpu.VMEM((_VOCAB,), jnp.float32),
            pltpu.VMEM((_CH,), jnp.int32),
            pltpu.VMEM((_CH,), jnp.int32),
            pltpu.VMEM((_CH,), jnp.int32),
            pltpu.VMEM((_CH,), jnp.int32),
            pltpu.VMEM((_L,), jnp.int32),
            pltpu.VMEM((4 * _CH,), jnp.float32),
            pltpu.VMEM((4 * _CH,), jnp.float32),
            pltpu.SemaphoreType.DMA,
            pltpu.SemaphoreType.DMA,
            pltpu.SemaphoreType.DMA,
            pltpu.SemaphoreType.DMA,
            pltpu.SemaphoreType.DMA,
            pltpu.SemaphoreType.DMA,
            pltpu.SemaphoreType.DMA,
        ],
    )
    def _sc_dim_dots(embT_hbm, tidx_hbm, cidx_hbm, nidx_hbm, parts_out,
                     ed, tq0, tq1, cq0, cq1, niv, av0, av1,
                     sem_e, st0, st1, sc0, sc1, so0, so1):
        wid = lax.axis_index("s") * _NC + lax.axis_index("c")
        pltpu.sync_copy(nidx_hbm, niv)
        nvals = niv[...]
        tqs, cqs, avs = (tq0, tq1), (cq0, cq1), (av0, av1)
        sts, scs, sos = (st0, st1), (sc0, sc1), (so0, so1)

        def start_idx(k, buf):
            cp_t = pltpu.async_copy(
                tidx_hbm.at[pl.ds(k * _CH, _CH)], tqs[buf], sts[buf])
            cp_c = pltpu.async_copy(
                cidx_hbm.at[pl.ds(k * _CH, _CH)], cqs[buf], scs[buf])
            return cp_t, cp_c

        for r in range(_NROUND):
            d = r * _NW + wid
            pltpu.sync_copy(embT_hbm.at[d], ed)
            nd = plsc.load_gather(ed, [nvals])
            nb0 = jnp.broadcast_to(nd[0], (_L,))
            nb1 = jnp.broadcast_to(nd[1], (_L,))
            nb2 = jnp.broadcast_to(nd[2], (_L,))
            row_base = d * _ROWLEN

            pend_idx = [start_idx(0, 0), None]
            pend_out = [None, None]
            for k in range(_NCH):
                buf = k & 1
                if k + 1 < _NCH:
                    pend_idx[1 - buf] = start_idx(k + 1, 1 - buf)
                cp_t, cp_c = pend_idx[buf]
                cp_t.wait()
                cp_c.wait()
                if pend_out[buf] is not None:
                    pend_out[buf].wait()
                tqi, cqi, accv = tqs[buf], cqs[buf], avs[buf]

                def group(jj, _, tqi=tqi, cqi=cqi, accv=accv,
                          nb0=nb0, nb1=nb1, nb2=nb2):
                    js = [jj * 4 + u for u in range(4)]
                    tvs = [tqi[pl.ds(j * _L, _L)] for j in js]
                    cvs = [cqi[pl.ds(j * _L, _L)] for j in js]
                    tcols = [plsc.load_gather(ed, [tv]) for tv in tvs]
                    ccols = [plsc.load_gather(ed, [cv]) for cv in cvs]
                    for u, j in enumerate(js):
                        accv[pl.ds(j * _L, _L)] = tcols[u] * ccols[u]
                    for u, j in enumerate(js):
                        accv[pl.ds(_CH + j * _L, _L)] = tcols[u] * nb0
                        accv[pl.ds(2 * _CH + j * _L, _L)] = tcols[u] * nb1
                        accv[pl.ds(3 * _CH + j * _L, _L)] = tcols[u] * nb2
                    return _

                lax.fori_loop(0, _CH // (_L * 4), group, None)
                pend_out[buf] = pltpu.async_copy(
                    accv,
                    parts_out.at[pl.ds(row_base + k * 4 * _CH, 4 * _CH)],
                    sos[buf])
            for po in pend_out:
                if po is not None:
                    po.wait()

    return _sc_dim_dots


_TCROWS = _ROWLEN // 128


def _tc_body(parts_ref, out_ref, acc_ref):
    i = pl.program_id(0)
    x = parts_ref[...]
    part = x[0:_TCROWS]
    for k in range(1, 8):
        part = part + x[k * _TCROWS:(k + 1) * _TCROWS]

    @pl.when(i == 0)
    def _():
        acc_ref[...] = jnp.zeros((_TCROWS, 128), jnp.float32)

    acc_ref[...] += part

    @pl.when(i == _D // 8 - 1)
    def _():
        s = acc_ref[...]
        ch = (lax.broadcasted_iota(jnp.int32, (_TCROWS, 128), 0)
              // (_CH // 128)) % 4
        s = jnp.where(ch == 0, s, -s)
        ls = jnp.minimum(s, 0.0) - jnp.log1p(jnp.exp(-jnp.abs(s)))
        out_ref[0, 0] = -jnp.sum(ls) / _B


def _tc_loss(parts):
    return pl.pallas_call(
        _tc_body,
        grid=(_D // 8,),
        in_specs=[
            pl.BlockSpec((8 * _TCROWS, 128), lambda i: (i, 0)),
        ],
        out_specs=pl.BlockSpec(memory_space=pltpu.SMEM),
        out_shape=jax.ShapeDtypeStruct((1, 1), jnp.float32),
        scratch_shapes=[pltpu.VMEM((_TCROWS, 128), jnp.float32)],
    )(parts.reshape(_D * _TCROWS, 128))


def kernel(target_idx, context_idx, embeddings, neg_idx):
    nidx = jnp.concatenate(
        [neg_idx.astype(jnp.int32), jnp.zeros((_L - _NEG,), jnp.int32)])
    parts = _build_sc_dim_dots()(
        embeddings.T, target_idx.astype(jnp.int32),
        context_idx.astype(jnp.int32), nidx)
    acc = _tc_loss(parts)
    return acc[0, 0]


# speedup vs baseline: 1.0304x; 1.0060x over previous
import functools

import jax
import jax.numpy as jnp
from jax import lax
from jax.experimental import pallas as pl
from jax.experimental.pallas import tpu as pltpu
from jax.experimental.pallas import tpu_sc as plsc

_VOCAB = 100000
_D = 64
_B = 16384
_NEG = 3
_L = 16

_NC, _NS = 2, 16
_NW = _NC * _NS
_NROUND = _D // _NW
_CH = 2048
_NCH = _B // _CH
_ROWLEN = _B * 4


@functools.cache
def _build_sc_dim_dots():
    mesh = plsc.VectorSubcoreMesh(core_axis_name="c", subcore_axis_name="s")

    @functools.partial(
        pl.kernel,
        mesh=mesh,
        compiler_params=pltpu.CompilerParams(
            use_tc_tiling_on_sc=True, needs_layout_passes=False),
        out_type=jax.ShapeDtypeStruct((_D * _ROWLEN,), jnp.float32),
        scratch_types=[
            pltpu.VMEM((_VOCAB,), jnp.float32),
            pltpu.VMEM((_CH,), jnp.int32),
            pltpu.VMEM((_CH,), jnp.int32),
            pltpu.VMEM((_CH,), jnp.int32),
            pltpu.VMEM((_CH,), jnp.int32),
            pltpu.VMEM((_L,), jnp.int32),
            pltpu.VMEM((4 * _CH,), jnp.float32),
            pltpu.VMEM((4 * _CH,), jnp.float32),
            pltpu.SemaphoreType.DMA,
            pltpu.SemaphoreType.DMA,
            pltpu.SemaphoreType.DMA,
            pltpu.SemaphoreType.DMA,
            pltpu.SemaphoreType.DMA,
            pltpu.SemaphoreType.DMA,
            pltpu.SemaphoreType.DMA,
        ],
    )
    def _sc_dim_dots(embT_hbm, tidx_hbm, cidx_hbm, nidx_hbm, parts_out,
                     ed, tq0, tq1, cq0, cq1, niv, av0, av1,
                     sem_e, st0, st1, sc0, sc1, so0, so1):
        wid = lax.axis_index("s") * _NC + lax.axis_index("c")
        pltpu.sync_copy(nidx_hbm, niv)
        nvals = niv[...]
        tqs, cqs, avs = (tq0, tq1), (cq0, cq1), (av0, av1)
        sts, scs, sos = (st0, st1), (sc0, sc1), (so0, so1)

        def start_idx(k, buf):
            cp_t = pltpu.async_copy(
                tidx_hbm.at[pl.ds(k * _CH, _CH)], tqs[buf], sts[buf])
            cp_c = pltpu.async_copy(
                cidx_hbm.at[pl.ds(k * _CH, _CH)], cqs[buf], scs[buf])
            return cp_t, cp_c

        for r in range(_NROUND):
            d = r * _NW + wid
            pend_idx = [start_idx(0, 0), None]
            pltpu.sync_copy(embT_hbm.at[d], ed)
            nd = plsc.load_gather(ed, [nvals])
            nb0 = jnp.broadcast_to(nd[0], (_L,))
            nb1 = jnp.broadcast_to(nd[1], (_L,))
            nb2 = jnp.broadcast_to(nd[2], (_L,))
            row_base = d * _ROWLEN

            pend_out = [None, None]
            for k in range(_NCH):
                buf = k & 1
                if k + 1 < _NCH:
                    pend_idx[1 - buf] = start_idx(k + 1, 1 - buf)
                cp_t, cp_c = pend_idx[buf]
                cp_t.wait()
                cp_c.wait()
                if pend_out[buf] is not None:
                    pend_out[buf].wait()
                tqi, cqi, accv = tqs[buf], cqs[buf], avs[buf]

                def group(jj, _, tqi=tqi, cqi=cqi, accv=accv,
                          nb0=nb0, nb1=nb1, nb2=nb2):
                    js = [jj * 4 + u for u in range(4)]
                    tvs = [tqi[pl.ds(j * _L, _L)] for j in js]
                    cvs = [cqi[pl.ds(j * _L, _L)] for j in js]
                    tcols = [plsc.load_gather(ed, [tv]) for tv in tvs]
                    ccols = [plsc.load_gather(ed, [cv]) for cv in cvs]
                    for u, j in enumerate(js):
                        accv[pl.ds(j * _L, _L)] = tcols[u] * ccols[u]
                    for u, j in enumerate(js):
                        accv[pl.ds(_CH + j * _L, _L)] = tcols[u] * nb0
                        accv[pl.ds(2 * _CH + j * _L, _L)] = tcols[u] * nb1
                        accv[pl.ds(3 * _CH + j * _L, _L)] = tcols[u] * nb2
                    return _

                lax.fori_loop(0, _CH // (_L * 4), group, None)
                pend_out[buf] = pltpu.async_copy(
                    accv,
                    parts_out.at[pl.ds(row_base + k * 4 * _CH, 4 * _CH)],
                    sos[buf])
            for po in pend_out:
                if po is not None:
                    po.wait()

    return _sc_dim_dots


_TCROWS = _ROWLEN // 128


def _tc_body(parts_ref, out_ref, acc_ref):
    i = pl.program_id(0)
    x = parts_ref[...]
    part = x[0:_TCROWS]
    for k in range(1, 16):
        part = part + x[k * _TCROWS:(k + 1) * _TCROWS]

    @pl.when(i == 0)
    def _():
        acc_ref[...] = jnp.zeros((_TCROWS, 128), jnp.float32)

    acc_ref[...] += part

    @pl.when(i == _D // 16 - 1)
    def _():
        s = acc_ref[...]
        ch = (lax.broadcasted_iota(jnp.int32, (_TCROWS, 128), 0)
              // (_CH // 128)) % 4
        s = jnp.where(ch == 0, s, -s)
        ls = jnp.minimum(s, 0.0) - jnp.log1p(jnp.exp(-jnp.abs(s)))
        out_ref[0, 0] = -jnp.sum(ls) / _B


def _tc_loss(parts):
    return pl.pallas_call(
        _tc_body,
        grid=(_D // 16,),
        in_specs=[
            pl.BlockSpec((16 * _TCROWS, 128), lambda i: (i, 0)),
        ],
        out_specs=pl.BlockSpec(memory_space=pltpu.SMEM),
        out_shape=jax.ShapeDtypeStruct((1, 1), jnp.float32),
        scratch_shapes=[pltpu.VMEM((_TCROWS, 128), jnp.float32)],
    )(parts.reshape(_D * _TCROWS, 128))


def kernel(target_idx, context_idx, embeddings, neg_idx):
    nidx = jnp.concatenate(
        [neg_idx.astype(jnp.int32), jnp.zeros((_L - _NEG,), jnp.int32)])
    parts = _build_sc_dim_dots()(
        embeddings.T, target_idx.astype(jnp.int32),
        context_idx.astype(jnp.int32), nidx)
    acc = _tc_loss(parts)
    return acc[0, 0]


# speedup vs baseline: 1.1351x; 1.1015x over previous
import functools

import jax
import jax.numpy as jnp
from jax import lax
from jax.experimental import pallas as pl
from jax.experimental.pallas import tpu as pltpu
from jax.experimental.pallas import tpu_sc as plsc

_VOCAB = 100000
_D = 64
_B = 16384
_NEG = 3
_L = 16

_NC, _NS = 2, 16
_NW = _NC * _NS
_NROUND = _D // _NW
_CH = 2048
_NCH = _B // _CH
_ROWLEN = _B * 4


@functools.cache
def _build_sc_dim_dots():
    mesh = plsc.VectorSubcoreMesh(core_axis_name="c", subcore_axis_name="s")

    @functools.partial(
        pl.kernel,
        mesh=mesh,
        compiler_params=pltpu.CompilerParams(
            use_tc_tiling_on_sc=True, needs_layout_passes=False),
        out_type=jax.ShapeDtypeStruct((_D * _ROWLEN,), jnp.bfloat16),
        scratch_types=[
            pltpu.VMEM((_VOCAB,), jnp.float32),
            pltpu.VMEM((_CH,), jnp.int32),
            pltpu.VMEM((_CH,), jnp.int32),
            pltpu.VMEM((_CH,), jnp.int32),
            pltpu.VMEM((_CH,), jnp.int32),
            pltpu.VMEM((_L,), jnp.int32),
            pltpu.VMEM((4 * _CH,), jnp.bfloat16),
            pltpu.VMEM((4 * _CH,), jnp.bfloat16),
            pltpu.SemaphoreType.DMA,
            pltpu.SemaphoreType.DMA,
            pltpu.SemaphoreType.DMA,
            pltpu.SemaphoreType.DMA,
            pltpu.SemaphoreType.DMA,
            pltpu.SemaphoreType.DMA,
            pltpu.SemaphoreType.DMA,
        ],
    )
    def _sc_dim_dots(embT_hbm, tidx_hbm, cidx_hbm, nidx_hbm, parts_out,
                     ed, tq0, tq1, cq0, cq1, niv, av0, av1,
                     sem_e, st0, st1, sc0, sc1, so0, so1):
        wid = lax.axis_index("s") * _NC + lax.axis_index("c")
        pltpu.sync_copy(nidx_hbm, niv)
        nvals = niv[...]
        tqs, cqs, avs = (tq0, tq1), (cq0, cq1), (av0, av1)
        sts, scs, sos = (st0, st1), (sc0, sc1), (so0, so1)

        def start_idx(k, buf):
            cp_t = pltpu.async_copy(
                tidx_hbm.at[pl.ds(k * _CH, _CH)], tqs[buf], sts[buf])
            cp_c = pltpu.async_copy(
                cidx_hbm.at[pl.ds(k * _CH, _CH)], cqs[buf], scs[buf])
            return cp_t, cp_c

        for r in range(_NROUND):
            d = r * _NW + wid
            pend_idx = [start_idx(0, 0), None]
            pltpu.sync_copy(embT_hbm.at[d], ed)
            nd = plsc.load_gather(ed, [nvals])
            nb0 = jnp.broadcast_to(nd[0], (_L,))
            nb1 = jnp.broadcast_to(nd[1], (_L,))
            nb2 = jnp.broadcast_to(nd[2], (_L,))
            row_base = d * _ROWLEN

            pend_out = [None, None]
            for k in range(_NCH):
                buf = k & 1
                if k + 1 < _NCH:
                    pend_idx[1 - buf] = start_idx(k + 1, 1 - buf)
                cp_t, cp_c = pend_idx[buf]
                cp_t.wait()
                cp_c.wait()
                if pend_out[buf] is not None:
                    pend_out[buf].wait()
                tqi, cqi, accv = tqs[buf], cqs[buf], avs[buf]

                def group(jj, _, tqi=tqi, cqi=cqi, accv=accv,
                          nb0=nb0, nb1=nb1, nb2=nb2):
                    js = [jj * 4 + u for u in range(4)]
                    tvs = [tqi[pl.ds(j * _L, _L)] for j in js]
                    cvs = [cqi[pl.ds(j * _L, _L)] for j in js]
                    tcols = [plsc.load_gather(ed, [tv]) for tv in tvs]
                    ccols = [plsc.load_gather(ed, [cv]) for cv in cvs]
                    for u in (0, 2):
                        j = js[u]
                        for base, rhs in ((0, None), (_CH, nb0),
                                          (2 * _CH, nb1), (3 * _CH, nb2)):
                            a = tcols[u] * (ccols[u] if rhs is None else rhs)
                            b = tcols[u + 1] * (
                                ccols[u + 1] if rhs is None else rhs)
                            accv[pl.ds(base + j * _L, 2 * _L)] = plsc.pack(
                                a, b, format=plsc.PackFormat.INTERLEAVED)
                    return _

                lax.fori_loop(0, _CH // (_L * 4), group, None)
                pend_out[buf] = pltpu.async_copy(
                    accv,
                    parts_out.at[pl.ds(row_base + k * 4 * _CH, 4 * _CH)],
                    sos[buf])
            for po in pend_out:
                if po is not None:
                    po.wait()

    return _sc_dim_dots


_TCROWS = _ROWLEN // 128


def _tc_body(parts_ref, out_ref, acc_ref):
    i = pl.program_id(0)
    x = parts_ref[...]
    part = x[0:_TCROWS].astype(jnp.float32)
    for k in range(1, 16):
        part = part + x[k * _TCROWS:(k + 1) * _TCROWS].astype(jnp.float32)

    @pl.when(i == 0)
    def _():
        acc_ref[...] = jnp.zeros((_TCROWS, 128), jnp.float32)

    acc_ref[...] += part

    @pl.when(i == _D // 16 - 1)
    def _():
        s = acc_ref[...]
        ch = (lax.broadcasted_iota(jnp.int32, (_TCROWS, 128), 0)
              // (_CH // 128)) % 4
        s = jnp.where(ch == 0, s, -s)
        ls = jnp.minimum(s, 0.0) - jnp.log1p(jnp.exp(-jnp.abs(s)))
        out_ref[0, 0] = -jnp.sum(ls) / _B


def _tc_loss(parts):
    return pl.pallas_call(
        _tc_body,
        grid=(_D // 16,),
        in_specs=[
            pl.BlockSpec((16 * _TCROWS, 128), lambda i: (i, 0)),
        ],
        out_specs=pl.BlockSpec(memory_space=pltpu.SMEM),
        out_shape=jax.ShapeDtypeStruct((1, 1), jnp.float32),
        scratch_shapes=[pltpu.VMEM((_TCROWS, 128), jnp.float32)],
    )(parts.reshape(_D * _TCROWS, 128))


def kernel(target_idx, context_idx, embeddings, neg_idx):
    nidx = jnp.concatenate(
        [neg_idx.astype(jnp.int32), jnp.zeros((_L - _NEG,), jnp.int32)])
    parts = _build_sc_dim_dots()(
        embeddings.T, target_idx.astype(jnp.int32),
        context_idx.astype(jnp.int32), nidx)
    acc = _tc_loss(parts)
    return acc[0, 0]
